# baseline (speedup 1.0000x reference)
import jax
import jax.numpy as jnp
from jax.experimental import pallas as pl
from jax.experimental.pallas import tpu as pltpu


def _copy_kernel(eps_ref, out_ref, sem):
    copy = pltpu.make_async_copy(eps_ref, out_ref, sem)
    copy.start()
    copy.wait()


def kernel(batch_size, epsilon):
    n, d = epsilon.shape
    return pl.pallas_call(
        _copy_kernel,
        in_specs=[pl.BlockSpec(memory_space=pl.ANY)],
        out_specs=pl.BlockSpec(memory_space=pl.ANY),
        scratch_shapes=[pltpu.SemaphoreType.DMA],
        out_shape=jax.ShapeDtypeStruct((n, d), epsilon.dtype),
    )(epsilon)


# speedup vs baseline: 34.1177x; 34.1177x over previous
import jax
import jax.numpy as jnp
from jax.experimental import pallas as pl
from jax.experimental.pallas import tpu as pltpu

_ROWS_PER_BLOCK = 4096


def _copy_block(eps_ref, out_ref):
    out_ref[...] = eps_ref[...]


def kernel(batch_size, epsilon):
    n, d = epsilon.shape
    grid = (n // _ROWS_PER_BLOCK,)
    return pl.pallas_call(
        _copy_block,
        grid=grid,
        in_specs=[pl.BlockSpec((_ROWS_PER_BLOCK, d), lambda i: (i, 0))],
        out_specs=pl.BlockSpec((_ROWS_PER_BLOCK, d), lambda i: (i, 0)),
        out_shape=jax.ShapeDtypeStruct((n, d), epsilon.dtype),
        compiler_params=pltpu.CompilerParams(
            dimension_semantics=("arbitrary",),
        ),
    )(epsilon)


# speedup vs baseline: 42.2757x; 1.2391x over previous
import jax
import jax.numpy as jnp
from jax.experimental import pallas as pl
from jax.experimental.pallas import tpu as pltpu

_ROWS_PER_BLOCK = 8192


def _copy_block(eps_ref, out_ref):
    out_ref[...] = eps_ref[...]


def kernel(batch_size, epsilon):
    n, d = epsilon.shape
    grid = (n // _ROWS_PER_BLOCK,)
    return pl.pallas_call(
        _copy_block,
        grid=grid,
        in_specs=[pl.BlockSpec((_ROWS_PER_BLOCK, d), lambda i: (i, 0))],
        out_specs=pl.BlockSpec((_ROWS_PER_BLOCK, d), lambda i: (i, 0)),
        out_shape=jax.ShapeDtypeStruct((n, d), epsilon.dtype),
        compiler_params=pltpu.CompilerParams(
            dimension_semantics=("arbitrary",),
        ),
    )(epsilon)
